# Initial kernel scaffold; baseline (speedup 1.0000x reference)
import jax
import jax.numpy as jnp
from jax.experimental import pallas as pl


def kernel(x, tok_emb, pos_emb):
    raise NotImplementedError("write your pallas kernel here")


# speedup vs baseline: 1.0277x; 1.0277x over previous
import functools

import jax
import jax.numpy as jnp
from jax import lax
from jax.experimental import pallas as pl
from jax.experimental.pallas import tpu as pltpu
from jax.experimental.pallas import tpu_sc as plsc

_B, _S, _EMB = 4, 2048, 768
_N = _B * _S
_NW = 32
_SPW = _S // _NW
_SUB = 32
_NSUB = _SPW // _SUB
_STEPS = _B * _NSUB
_LANES = _EMB // 16
_NBUF = 3

_mesh = plsc.VectorSubcoreMesh(core_axis_name="c", subcore_axis_name="s")


@functools.partial(
    pl.kernel,
    out_type=jax.ShapeDtypeStruct((_N, _EMB), jnp.float32),
    mesh=_mesh,
    scratch_types=[
        pltpu.VMEM((_B * _SPW,), jnp.int32),
        pltpu.VMEM((_SPW, _EMB), jnp.float32),
        pltpu.VMEM((_SUB, _EMB), jnp.float32),
        pltpu.VMEM((_SUB, _EMB), jnp.float32),
        pltpu.VMEM((_SUB, _EMB), jnp.float32),
        pltpu.SemaphoreType.DMA,
        pltpu.SemaphoreType.DMA,
        pltpu.SemaphoreType.DMA,
        pltpu.SemaphoreType.DMA,
        pltpu.SemaphoreType.DMA,
        pltpu.SemaphoreType.DMA,
    ],
)
def _embed(xt_hbm, tok_hbm, pos_hbm, out_hbm,
           idx_v, pos_v, buf0, buf1, buf2,
           g0, g1, g2, s0, s1, s2):
    wid = lax.axis_index("s") * 2 + lax.axis_index("c")
    s_base = wid * _SPW

    bufs = (buf0, buf1, buf2)
    gsems = (g0, g1, g2)
    ssems = (s0, s1, s2)

    pltpu.sync_copy(xt_hbm.at[pl.ds(wid * (_B * _SPW), _B * _SPW)], idx_v)
    pltpu.sync_copy(pos_hbm.at[pl.ds(s_base, _SPW), :], pos_v)

    def out_slice(k):
        b, sub = divmod(k, _NSUB)
        return out_hbm.at[pl.ds(b * _S + s_base + sub * _SUB, _SUB), :]

    def gather_start(k):
        src = tok_hbm.at[idx_v.at[pl.ds(k * _SUB, _SUB)]]
        pltpu.async_copy(src, bufs[k % _NBUF], gsems[k % _NBUF])

    def gather_wait(k):
        src = tok_hbm.at[idx_v.at[pl.ds(k * _SUB, _SUB)]]
        pltpu.make_async_copy(src, bufs[k % _NBUF], gsems[k % _NBUF]).wait()

    def store_start(k):
        pltpu.async_copy(bufs[k % _NBUF], out_slice(k), ssems[k % _NBUF])

    def store_wait(k):
        pltpu.make_async_copy(bufs[k % _NBUF], out_slice(k), ssems[k % _NBUF]).wait()

    gather_start(0)
    gather_start(1)
    for k in range(_STEPS):
        if k + 2 < _STEPS:
            if k - 1 >= 0:
                store_wait(k - 1)
            gather_start(k + 2)
        gather_wait(k)

        buf = bufs[k % _NBUF]
        pos_off = (k % _NSUB) * _SUB

        def add_row(r, _):
            for j in range(_LANES):
                col = pl.ds(j * 16, 16)
                buf[r, col] = buf[r, col] + pos_v[pos_off + r, col]
            return 0

        lax.fori_loop(0, _SUB, add_row, 0)
        store_start(k)

    for k in range(_STEPS - _NBUF, _STEPS):
        store_wait(k)


def kernel(x, tok_emb, pos_emb):
    xt = (
        x.astype(jnp.int32)
        .reshape(_B, _NW, _SPW)
        .transpose(1, 0, 2)
        .reshape(_N)
    )
    out = _embed(xt, tok_emb, pos_emb)
    return out.reshape(_B, _S, _EMB)


# speedup vs baseline: 1.0967x; 1.0672x over previous
import functools

import jax
import jax.numpy as jnp
from jax import lax
from jax.experimental import pallas as pl
from jax.experimental.pallas import tpu as pltpu
from jax.experimental.pallas import tpu_sc as plsc

_B, _S, _EMB = 4, 2048, 768
_N = _B * _S
_NW = 32
_SPW = _S // _NW
_SUB = 32
_NSUB = _SPW // _SUB
_STEPS = _B * _NSUB
_LANES = _EMB // 16
_NBUF = 3

_mesh = plsc.VectorSubcoreMesh(core_axis_name="c", subcore_axis_name="s")


@functools.partial(
    pl.kernel,
    out_type=jax.ShapeDtypeStruct((_N, _EMB), jnp.float32),
    mesh=_mesh,
    scratch_types=[
        pltpu.VMEM((_B * _SPW,), jnp.int32),
        pltpu.VMEM((_SPW, _EMB), jnp.float32),
        pltpu.VMEM((_SUB, _EMB), jnp.float32),
        pltpu.VMEM((_SUB, _EMB), jnp.float32),
        pltpu.VMEM((_SUB, _EMB), jnp.float32),
        pltpu.SemaphoreType.DMA,
        pltpu.SemaphoreType.DMA,
        pltpu.SemaphoreType.DMA,
        pltpu.SemaphoreType.DMA,
        pltpu.SemaphoreType.DMA,
        pltpu.SemaphoreType.DMA,
    ],
)
def _embed(xt_hbm, tok_hbm, pos_hbm, out_hbm,
           idx_v, pos_v, buf0, buf1, buf2,
           g0, g1, g2, s0, s1, s2):
    wid = lax.axis_index("s") * 2 + lax.axis_index("c")
    s_base = wid * _SPW

    bufs = (buf0, buf1, buf2)
    gsems = (g0, g1, g2)
    ssems = (s0, s1, s2)

    pltpu.sync_copy(xt_hbm.at[pl.ds(wid * (_B * _SPW), _B * _SPW)], idx_v)
    pltpu.sync_copy(pos_hbm.at[pl.ds(s_base, _SPW), :], pos_v)

    def out_slice(k):
        b, sub = divmod(k, _NSUB)
        return out_hbm.at[pl.ds(b * _S + s_base + sub * _SUB, _SUB), :]

    def gather_start(k):
        src = tok_hbm.at[idx_v.at[pl.ds(k * _SUB, _SUB)]]
        pltpu.async_copy(src, bufs[k % _NBUF], gsems[k % _NBUF])

    def gather_wait(k):
        src = tok_hbm.at[idx_v.at[pl.ds(k * _SUB, _SUB)]]
        pltpu.make_async_copy(src, bufs[k % _NBUF], gsems[k % _NBUF]).wait()

    def store_start(k):
        pltpu.async_copy(bufs[k % _NBUF], out_slice(k), ssems[k % _NBUF])

    def store_wait(k):
        pltpu.make_async_copy(bufs[k % _NBUF], out_slice(k), ssems[k % _NBUF]).wait()

    gather_start(0)
    gather_start(1)
    for k in range(_STEPS):
        if k + 2 < _STEPS:
            if k - 1 >= 0:
                store_wait(k - 1)
            gather_start(k + 2)
        gather_wait(k)

        buf = bufs[k % _NBUF]
        pos_off = (k % _NSUB) * _SUB

        def add_row(r, _):
            for j in range(_LANES):
                col = pl.ds(j * 16, 16)
                plsc.addupdate(buf.at[r, col], pos_v[pos_off + r, col])
            return 0

        lax.fori_loop(0, _SUB, add_row, 0)
        store_start(k)

    for k in range(_STEPS - _NBUF, _STEPS):
        store_wait(k)


def kernel(x, tok_emb, pos_emb):
    xt = (
        x.astype(jnp.int32)
        .reshape(_B, _NW, _SPW)
        .transpose(1, 0, 2)
        .reshape(_N)
    )
    out = _embed(xt, tok_emb, pos_emb)
    return out.reshape(_B, _S, _EMB)


# speedup vs baseline: 1.1223x; 1.0233x over previous
import functools

import jax
import jax.numpy as jnp
from jax import lax
from jax.experimental import pallas as pl
from jax.experimental.pallas import tpu as pltpu
from jax.experimental.pallas import tpu_sc as plsc

_B, _S, _EMB = 4, 2048, 768
_N = _B * _S
_NW = 32
_SPW = _S // _NW
_SUB = 32
_NSUB = _SPW // _SUB
_STEPS = _B * _NSUB
_LANES = _EMB // 16
_NBUF = 3

_mesh = plsc.VectorSubcoreMesh(core_axis_name="c", subcore_axis_name="s")


@functools.partial(
    pl.kernel,
    out_type=jax.ShapeDtypeStruct((_N, _EMB), jnp.float32),
    mesh=_mesh,
    scratch_types=[
        pltpu.VMEM((_B * _SPW,), jnp.int32),
        pltpu.VMEM((_SPW, _EMB), jnp.float32),
        pltpu.VMEM((_SUB, _EMB), jnp.float32),
        pltpu.VMEM((_SUB, _EMB), jnp.float32),
        pltpu.VMEM((_SUB, _EMB), jnp.float32),
        pltpu.SemaphoreType.DMA,
        pltpu.SemaphoreType.DMA,
        pltpu.SemaphoreType.DMA,
        pltpu.SemaphoreType.DMA,
        pltpu.SemaphoreType.DMA,
        pltpu.SemaphoreType.DMA,
        pltpu.SemaphoreType.DMA,
    ],
)
def _embed(x_hbm, tok_hbm, pos_hbm, out_hbm,
           idx_v, pos_v, buf0, buf1, buf2,
           g0, g1, g2, s0, s1, s2, psem):
    wid = lax.axis_index("s") * 2 + lax.axis_index("c")
    s_base = wid * _SPW

    bufs = (buf0, buf1, buf2)
    gsems = (g0, g1, g2)
    ssems = (s0, s1, s2)

    pos_src = pos_hbm.at[pl.ds(s_base, _SPW), :]
    pltpu.async_copy(pos_src, pos_v, psem)
    pltpu.sync_copy(x_hbm.at[pl.ds(wid * (_B * _SPW), _B * _SPW)], idx_v)

    def idx_slice(k):
        return idx_v.at[pl.ds(k * _SUB, _SUB)]

    def out_slice(k):
        b, sub = divmod(k, _NSUB)
        return out_hbm.at[pl.ds(b * _S + s_base + sub * _SUB, _SUB), :]

    def gather_start(k):
        pltpu.async_copy(tok_hbm.at[idx_slice(k)], bufs[k % _NBUF], gsems[k % _NBUF])

    def gather_wait(k):
        pltpu.make_async_copy(
            tok_hbm.at[idx_slice(k)], bufs[k % _NBUF], gsems[k % _NBUF]
        ).wait()

    def store_start(k):
        pltpu.async_copy(bufs[k % _NBUF], out_slice(k), ssems[k % _NBUF])

    def store_wait(k):
        pltpu.make_async_copy(bufs[k % _NBUF], out_slice(k), ssems[k % _NBUF]).wait()

    gather_start(0)
    gather_start(1)
    for k in range(_STEPS):
        if k + 2 < _STEPS:
            if k - 1 >= 0:
                store_wait(k - 1)
            gather_start(k + 2)
        gather_wait(k)
        if k == 0:
            pltpu.make_async_copy(pos_src, pos_v, psem).wait()

        buf = bufs[k % _NBUF]
        pos_off = (k % _NSUB) * _SUB

        def add_row(r, _):
            for j in range(_LANES):
                col = pl.ds(j * 16, 16)
                plsc.addupdate(buf.at[r, col], pos_v[pos_off + r, col])
            return 0

        lax.fori_loop(0, _SUB, add_row, 0)
        store_start(k)

    for k in range(_STEPS - _NBUF, _STEPS):
        store_wait(k)


def kernel(x, tok_emb, pos_emb):
    xt = (
        x.astype(jnp.int32)
        .reshape(_B, _NW, _SPW)
        .transpose(1, 0, 2)
        .reshape(_N)
    )
    out = _embed(xt, tok_emb, pos_emb)
    return out.reshape(_B, _S, _EMB)
